# Initial kernel scaffold; baseline (speedup 1.0000x reference)
import jax
import jax.numpy as jnp
from jax.experimental import pallas as pl


def kernel(x, edge_index, W1, b1, W2, b2, Wp1, bp1, Wp2, bp2):
    raise NotImplementedError("write your pallas kernel here")


# speedup vs baseline: 20.2049x; 20.2049x over previous
import functools

import jax
import jax.numpy as jnp
from jax import lax
from jax.experimental import pallas as pl
from jax.experimental.pallas import tpu as pltpu
from jax.experimental.pallas import tpu_sc as plsc

N = 10000
NP = 10240
E = 320000
IN_DIM = 128
HID = 64
OUT_DIM = 32

NC = 2
NS = 16
NW = NC * NS
EPW = E // NW
K = 80
CH = EPW // K
RPT = NP // NS

_f32 = jnp.float32


@functools.cache
def _make_prop(d):
    mesh = plsc.VectorSubcoreMesh(core_axis_name="c", subcore_axis_name="s")

    @functools.partial(
        pl.kernel,
        out_type=jax.ShapeDtypeStruct((NC, NP, d), _f32),
        mesh=mesh,
        compiler_params=pltpu.CompilerParams(use_tc_tiling_on_sc=False),
        scratch_types=[
            pltpu.VMEM((CH, K), jnp.int32),
            pltpu.VMEM((CH, K), jnp.int32),
            pltpu.VMEM((K, d), _f32),
            pltpu.VMEM_SHARED((NP, d), _f32),
            pltpu.SemaphoreType.DMA,
        ],
    )
    def prop(hs_hbm, src_hbm, dst_hbm, out_hbm, src_v, dst_v, msg_v, acc_sh, sem):
        cid = lax.axis_index("c")
        sid = lax.axis_index("s")
        wid = sid * NC + cid
        base = sid * RPT

        pltpu.sync_copy(hs_hbm.at[pl.ds(base, RPT)], acc_sh.at[pl.ds(base, RPT)])
        pltpu.sync_copy(src_hbm.at[wid], src_v)
        pltpu.sync_copy(dst_hbm.at[wid], dst_v)
        plsc.subcore_barrier()

        def chunk(j, carry):
            pltpu.async_copy(hs_hbm.at[src_v.at[j]], msg_v, sem).wait()
            pltpu.sync_copy(msg_v, acc_sh.at[dst_v.at[j]], add=True)
            return carry

        lax.fori_loop(0, CH, chunk, 0)

        plsc.subcore_barrier()
        pltpu.sync_copy(
            acc_sh.at[pl.ds(base, RPT)], out_hbm.at[cid].at[pl.ds(base, RPT)]
        )

    return prop


RB = 1024
GRID = NP // RB


def _dinv(degp):
    deg = degp[0, :, 0:1] + degp[1, :, 0:1] - 1.0
    return lax.rsqrt(deg)


def _tc1_body(x_ref, w1_ref, degp_ref, hs1_ref):
    dinv = _dinv(degp_ref[...])
    h = jnp.dot(x_ref[...], w1_ref[...], preferred_element_type=_f32)
    hs1_ref[...] = h * dinv


def _tc2_body(acc_ref, hs1_ref, degp_ref, w2_ref, b1_ref, hs2_ref):
    dinv = _dinv(degp_ref[...])
    acc = acc_ref[0] + acc_ref[1] - hs1_ref[...]
    h1 = jnp.maximum(acc * dinv + b1_ref[...], 0.0)
    h2 = jnp.dot(h1, w2_ref[...], preferred_element_type=_f32)
    hs2_ref[...] = h2 * dinv


def _tc3_body(acc_ref, hs2_ref, degp_ref, b2_ref, wp1_ref, bp1_ref,
              wp2_ref, bp2_ref, h2_ref, nw_ref):
    dinv = _dinv(degp_ref[...])
    acc = acc_ref[0] + acc_ref[1] - hs2_ref[...]
    h2 = acc * dinv + b2_ref[...]
    h2_ref[...] = h2
    p = jnp.maximum(
        jnp.dot(h2, wp1_ref[...], preferred_element_type=_f32) + bp1_ref[...], 0.0
    )
    z = jnp.dot(p, wp2_ref[...], preferred_element_type=_f32) + bp2_ref[...]
    nw_ref[...] = jax.nn.sigmoid(z)


def _row_spec(d):
    return pl.BlockSpec((RB, d), lambda i: (i, 0))


def _full_spec(shape):
    return pl.BlockSpec(shape, lambda i: tuple(0 for _ in shape))


_degp_spec = pl.BlockSpec((2, RB, 16), lambda i: (0, i, 0))
_acc_spec = lambda d: pl.BlockSpec((2, RB, d), lambda i: (0, i, 0))


def _tc1(x, w1, degp):
    return pl.pallas_call(
        _tc1_body,
        grid=(GRID,),
        in_specs=[_row_spec(IN_DIM), _full_spec((IN_DIM, HID)), _degp_spec],
        out_specs=_row_spec(HID),
        out_shape=jax.ShapeDtypeStruct((NP, HID), _f32),
    )(x, w1, degp)


def _tc2(acc1, hs1, degp, w2, b1):
    return pl.pallas_call(
        _tc2_body,
        grid=(GRID,),
        in_specs=[
            _acc_spec(HID), _row_spec(HID), _degp_spec,
            _full_spec((HID, OUT_DIM)), _full_spec((1, HID)),
        ],
        out_specs=_row_spec(OUT_DIM),
        out_shape=jax.ShapeDtypeStruct((NP, OUT_DIM), _f32),
    )(acc1, hs1, degp, w2, b1)


def _tc3(acc2, hs2, degp, b2, wp1, bp1, wp2p, bp2):
    return pl.pallas_call(
        _tc3_body,
        grid=(GRID,),
        in_specs=[
            _acc_spec(OUT_DIM), _row_spec(OUT_DIM), _degp_spec,
            _full_spec((1, OUT_DIM)), _full_spec((OUT_DIM, HID)),
            _full_spec((1, HID)), _full_spec((HID, 8)), _full_spec((1, 1)),
        ],
        out_specs=[_row_spec(OUT_DIM), _row_spec(8)],
        out_shape=[
            jax.ShapeDtypeStruct((NP, OUT_DIM), _f32),
            jax.ShapeDtypeStruct((NP, 8), _f32),
        ],
    )(acc2, hs2, degp, b2, wp1, bp1, wp2p, bp2)


def kernel(x, edge_index, W1, b1, W2, b2, Wp1, bp1, Wp2, bp2):
    ei = edge_index.astype(jnp.int32)
    src3 = ei[0].reshape(NW, CH, K)
    dst3 = ei[1].reshape(NW, CH, K)
    xp = jnp.pad(x, ((0, NP - N), (0, 0)))

    ones16 = jnp.ones((NP, 16), _f32)
    degp = _make_prop(16)(ones16, src3, dst3)

    hs1 = _tc1(xp, W1, degp)
    acc1 = _make_prop(HID)(hs1, src3, dst3)
    hs2 = _tc2(acc1, hs1, degp, W2, b1.reshape(1, HID))
    acc2 = _make_prop(OUT_DIM)(hs2, src3, dst3)

    wp2p = jnp.pad(Wp2, ((0, 0), (0, 7)))
    h2, nw8 = _tc3(acc2, hs2, degp, b2.reshape(1, OUT_DIM),
                   Wp1, bp1.reshape(1, HID), wp2p, bp2.reshape(1, 1))
    return (h2[:N], nw8[:N, :1])
